# baseline (speedup 1.0000x reference)
import jax
import jax.numpy as jnp
from jax.experimental import pallas as pl
from jax.experimental.pallas import tpu as pltpu

_LEGS = (2000, 2000, 2000, 2000, 2000)


def _gelu_exact(x):
    return 0.5 * x * (1.0 + jax.lax.erf(x * 0.7071067811865476))


def _double_gelu(x):
    return _gelu_exact(_gelu_exact(x))


def _make_pipelined_kernel(legs):
    n_legs = len(legs)
    offs = [sum(legs[:i]) for i in range(n_legs)]

    def body(x_hbm, o_hbm, *scratch):
        bufs = scratch[:n_legs]
        in_sems = scratch[n_legs:2 * n_legs]
        out_sems = scratch[2 * n_legs:]
        loads = [
            pltpu.make_async_copy(
                x_hbm.at[pl.ds(offs[i], legs[i]), :], bufs[i], in_sems[i])
            for i in range(n_legs)
        ]
        stores = [
            pltpu.make_async_copy(
                bufs[i], o_hbm.at[pl.ds(offs[i], legs[i]), :], out_sems[i])
            for i in range(n_legs)
        ]
        lookahead = min(2, n_legs)
        for ld in loads[:lookahead]:
            ld.start()
        for i in range(n_legs):
            loads[i].wait()
            bufs[i][...] = _double_gelu(bufs[i][...])
            if i + lookahead < n_legs:
                loads[i + lookahead].start()
            stores[i].start()
        for st in stores:
            st.wait()

    return body


def kernel(embs, edge_index, batch_size):
    del edge_index, batch_size
    n, d = embs.shape
    if n == sum(_LEGS) and all(leg % 8 == 0 for leg in _LEGS):
        legs = _LEGS
    else:
        legs = (n,)
    body = _make_pipelined_kernel(legs)
    scratch = (
        [pltpu.VMEM((leg, d), embs.dtype) for leg in legs]
        + [pltpu.SemaphoreType.DMA] * (2 * len(legs))
    )
    return pl.pallas_call(
        body,
        in_specs=[pl.BlockSpec(memory_space=pltpu.MemorySpace.HBM)],
        out_specs=pl.BlockSpec(memory_space=pltpu.MemorySpace.HBM),
        out_shape=jax.ShapeDtypeStruct((n, d), embs.dtype),
        scratch_shapes=scratch,
    )(embs)


# speedup vs baseline: 1.2694x; 1.2694x over previous
import jax
import jax.numpy as jnp
from jax.experimental import pallas as pl
from jax.experimental.pallas import tpu as pltpu

_LEGS = (2000, 2000, 2000, 2000, 2000)


def _gelu_exact(x):
    return 0.5 * x * (1.0 + jax.lax.erf(x * 0.7071067811865476))


def _double_gelu(x):
    return _gelu_exact(_gelu_exact(x))


def _make_pipelined_kernel(legs):
    n_legs = len(legs)
    offs = [sum(legs[:i]) for i in range(n_legs)]

    def body(x_hbm, o_hbm, *scratch):
        bufs = scratch[:n_legs]
        in_sems = scratch[n_legs:2 * n_legs]
        out_sems = scratch[2 * n_legs:]
        loads = [
            pltpu.make_async_copy(
                x_hbm.at[pl.ds(offs[i], legs[i]), :], bufs[i], in_sems[i])
            for i in range(n_legs)
        ]
        stores = [
            pltpu.make_async_copy(
                bufs[i], o_hbm.at[pl.ds(offs[i], legs[i]), :], out_sems[i])
            for i in range(n_legs)
        ]
        for ld in loads:
            ld.start()
        for i in range(n_legs):
            loads[i].wait()
            bufs[i][...] = _double_gelu(bufs[i][...])
            stores[i].start()
        for st in stores:
            st.wait()

    return body


def kernel(embs, edge_index, batch_size):
    del edge_index, batch_size
    n, d = embs.shape
    if n == sum(_LEGS) and all(leg % 8 == 0 for leg in _LEGS):
        legs = _LEGS
    else:
        legs = (n,)
    body = _make_pipelined_kernel(legs)
    scratch = (
        [pltpu.VMEM((leg, d), embs.dtype) for leg in legs]
        + [pltpu.SemaphoreType.DMA] * (2 * len(legs))
    )
    return pl.pallas_call(
        body,
        in_specs=[pl.BlockSpec(memory_space=pltpu.MemorySpace.HBM)],
        out_specs=pl.BlockSpec(memory_space=pltpu.MemorySpace.HBM),
        out_shape=jax.ShapeDtypeStruct((n, d), embs.dtype),
        scratch_shapes=scratch,
    )(embs)


# speedup vs baseline: 1.2822x; 1.0101x over previous
import jax
import jax.numpy as jnp
from jax.experimental import pallas as pl
from jax.experimental.pallas import tpu as pltpu

_LEGS = (1672, 1672, 1664, 1664, 1664, 1664)


def _gelu_exact(x):
    return 0.5 * x * (1.0 + jax.lax.erf(x * 0.7071067811865476))


def _double_gelu(x):
    return _gelu_exact(_gelu_exact(x))


def _make_pipelined_kernel(legs):
    n_legs = len(legs)
    offs = [sum(legs[:i]) for i in range(n_legs)]

    def body(x_hbm, o_hbm, *scratch):
        bufs = scratch[:n_legs]
        in_sems = scratch[n_legs:2 * n_legs]
        out_sems = scratch[2 * n_legs:]
        loads = [
            pltpu.make_async_copy(
                x_hbm.at[pl.ds(offs[i], legs[i]), :], bufs[i], in_sems[i])
            for i in range(n_legs)
        ]
        stores = [
            pltpu.make_async_copy(
                bufs[i], o_hbm.at[pl.ds(offs[i], legs[i]), :], out_sems[i])
            for i in range(n_legs)
        ]
        for ld in loads:
            ld.start()
        for i in range(n_legs):
            loads[i].wait()
            bufs[i][...] = _double_gelu(bufs[i][...])
            stores[i].start()
        for st in stores:
            st.wait()

    return body


def kernel(embs, edge_index, batch_size):
    del edge_index, batch_size
    n, d = embs.shape
    if n == sum(_LEGS) and all(leg % 8 == 0 for leg in _LEGS):
        legs = _LEGS
    else:
        legs = (n,)
    body = _make_pipelined_kernel(legs)
    scratch = (
        [pltpu.VMEM((leg, d), embs.dtype) for leg in legs]
        + [pltpu.SemaphoreType.DMA] * (2 * len(legs))
    )
    return pl.pallas_call(
        body,
        in_specs=[pl.BlockSpec(memory_space=pltpu.MemorySpace.HBM)],
        out_specs=pl.BlockSpec(memory_space=pltpu.MemorySpace.HBM),
        out_shape=jax.ShapeDtypeStruct((n, d), embs.dtype),
        scratch_shapes=scratch,
    )(embs)


# speedup vs baseline: 1.2899x; 1.0060x over previous
import jax
import jax.numpy as jnp
from jax.experimental import pallas as pl
from jax.experimental.pallas import tpu as pltpu

_LEGS = (1432, 1432, 1432, 1432, 1424, 1424, 1424)


def _gelu_exact(x):
    return 0.5 * x * (1.0 + jax.lax.erf(x * 0.7071067811865476))


def _double_gelu(x):
    return _gelu_exact(_gelu_exact(x))


def _make_pipelined_kernel(legs):
    n_legs = len(legs)
    offs = [sum(legs[:i]) for i in range(n_legs)]

    def body(x_hbm, o_hbm, *scratch):
        bufs = scratch[:n_legs]
        in_sems = scratch[n_legs:2 * n_legs]
        out_sems = scratch[2 * n_legs:]
        loads = [
            pltpu.make_async_copy(
                x_hbm.at[pl.ds(offs[i], legs[i]), :], bufs[i], in_sems[i])
            for i in range(n_legs)
        ]
        stores = [
            pltpu.make_async_copy(
                bufs[i], o_hbm.at[pl.ds(offs[i], legs[i]), :], out_sems[i])
            for i in range(n_legs)
        ]
        for ld in loads:
            ld.start()
        for i in range(n_legs):
            loads[i].wait()
            bufs[i][...] = _double_gelu(bufs[i][...])
            stores[i].start()
        for st in stores:
            st.wait()

    return body


def kernel(embs, edge_index, batch_size):
    del edge_index, batch_size
    n, d = embs.shape
    if n == sum(_LEGS) and all(leg % 8 == 0 for leg in _LEGS):
        legs = _LEGS
    else:
        legs = (n,)
    body = _make_pipelined_kernel(legs)
    scratch = (
        [pltpu.VMEM((leg, d), embs.dtype) for leg in legs]
        + [pltpu.SemaphoreType.DMA] * (2 * len(legs))
    )
    return pl.pallas_call(
        body,
        in_specs=[pl.BlockSpec(memory_space=pltpu.MemorySpace.HBM)],
        out_specs=pl.BlockSpec(memory_space=pltpu.MemorySpace.HBM),
        out_shape=jax.ShapeDtypeStruct((n, d), embs.dtype),
        scratch_shapes=scratch,
    )(embs)
